# Initial kernel scaffold; baseline (speedup 1.0000x reference)
import jax
import jax.numpy as jnp
from jax.experimental import pallas as pl


def kernel(z, edge_index, W_src, W_dst, attn_a, gat_bias, fc_W, fc_b):
    raise NotImplementedError("write your pallas kernel here")


# speedup vs baseline: 6.3421x; 6.3421x over previous
import jax
import jax.numpy as jnp
from jax.experimental import pallas as pl

N = 10000
E = 160000
D_IN = 256
H = 4
D_OUT = 128

BN = 400
BE = 1000


def _proj_body(z_ref, ws_ref, wd_ref, u_ref, v_ref):
    zb = z_ref[...]
    u_ref[...] = jnp.dot(zb, ws_ref[...], preferred_element_type=jnp.float32)
    v_ref[...] = jnp.dot(zb, wd_ref[...], preferred_element_type=jnp.float32)


_proj = pl.pallas_call(
    _proj_body,
    grid=(N // BN,),
    in_specs=[
        pl.BlockSpec((BN, D_IN), lambda i: (i, 0)),
        pl.BlockSpec((D_IN, H * D_OUT), lambda i: (0, 0)),
        pl.BlockSpec((D_IN, H * D_OUT), lambda i: (0, 0)),
    ],
    out_specs=[
        pl.BlockSpec((BN, H * D_OUT), lambda i: (i, 0)),
        pl.BlockSpec((BN, H * D_OUT), lambda i: (i, 0)),
    ],
    out_shape=[
        jax.ShapeDtypeStruct((N, H * D_OUT), jnp.float32),
        jax.ShapeDtypeStruct((N, H * D_OUT), jnp.float32),
    ],
)


def _edge_body(u_ref, v_ref, a_ref, msg_ref, ex_ref):
    u = u_ref[...]
    x = u + v_ref[...]
    t = 0.6 * x + 0.4 * jnp.abs(x)
    t4 = t.reshape(BE, H, D_OUT)
    logits = jnp.sum(t4 * a_ref[...][None, :, :], axis=-1)
    ex = jnp.exp(logits)
    ex_ref[...] = ex
    msg = u.reshape(BE, H, D_OUT) * ex[:, :, None]
    msg_ref[...] = msg.reshape(BE, H * D_OUT)


_edge = pl.pallas_call(
    _edge_body,
    grid=(E // BE,),
    in_specs=[
        pl.BlockSpec((BE, H * D_OUT), lambda i: (i, 0)),
        pl.BlockSpec((BE, H * D_OUT), lambda i: (i, 0)),
        pl.BlockSpec((H, D_OUT), lambda i: (0, 0)),
    ],
    out_specs=[
        pl.BlockSpec((BE, H * D_OUT), lambda i: (i, 0)),
        pl.BlockSpec((BE, H), lambda i: (i, 0)),
    ],
    out_shape=[
        jax.ShapeDtypeStruct((E, H * D_OUT), jnp.float32),
        jax.ShapeDtypeStruct((E, H), jnp.float32),
    ],
)


def _fc_body(r_ref, d_ref, b_ref, w_ref, fb_ref, o_ref):
    acc = jnp.zeros((BN, D_OUT), jnp.float32) + fb_ref[...]
    den = jnp.maximum(d_ref[...], 1e-30)
    for h in range(H):
        feat = (r_ref[...][:, h * D_OUT:(h + 1) * D_OUT] / den[:, h:h + 1]
                + b_ref[...][h][None, :])
        acc = acc + jnp.dot(feat, w_ref[h], preferred_element_type=jnp.float32)
    o_ref[...] = jnp.where(acc > 0, acc, 0.2 * acc)


_fc = pl.pallas_call(
    _fc_body,
    grid=(N // BN,),
    in_specs=[
        pl.BlockSpec((BN, H * D_OUT), lambda i: (i, 0)),
        pl.BlockSpec((BN, H), lambda i: (i, 0)),
        pl.BlockSpec((H, D_OUT), lambda i: (0, 0)),
        pl.BlockSpec((H, D_OUT, D_OUT), lambda i: (0, 0, 0)),
        pl.BlockSpec((1, D_OUT), lambda i: (0, 0)),
    ],
    out_specs=pl.BlockSpec((BN, D_OUT), lambda i: (i, 0)),
    out_shape=jax.ShapeDtypeStruct((N, D_OUT), jnp.float32),
)


def kernel(z, edge_index, W_src, W_dst, attn_a, gat_bias, fc_W, fc_b):
    src = edge_index[0]
    dst = edge_index[1]
    U, V = _proj(z, W_src, W_dst)
    msg, ex = _edge(U[src], V[dst], attn_a)
    den = jax.ops.segment_sum(ex, dst, num_segments=N)
    rstU = jax.ops.segment_sum(msg, dst, num_segments=N)
    out = _fc(rstU, den, gat_bias.reshape(H, D_OUT),
              fc_W.reshape(H, D_OUT, D_OUT), fc_b.reshape(1, D_OUT))
    return out
